# Initial kernel scaffold; baseline (speedup 1.0000x reference)
import jax
import jax.numpy as jnp
from jax import lax
from jax.experimental import pallas as pl
from jax.experimental.pallas import tpu as pltpu
from jax.experimental.pallas import tpu_sc as plsc

D = 64
L = 16
NW = 32
CHUNK = 128
SEQ = 200


def _rsqrt(x):
    xi = lax.bitcast_convert_type(x, jnp.int32)
    yi = jnp.int32(0x5F3759DF) - lax.shift_right_arithmetic(xi, jnp.int32(1))
    y = lax.bitcast_convert_type(yi, jnp.float32)
    half_x = x * 0.5
    for _ in range(3):
        y = y * (1.5 - half_x * y * y)
    return y


def _body(tok_hbm, idx_hbm, pos_hbm, g_hbm, b_hbm, out_hbm,
          pos_v, g_v, b_v, idx0, idx1, rows0, rows1, sem0, sem1):
    wid = lax.axis_index("s") * 2 + lax.axis_index("c")
    n_chunks = idx_hbm.shape[0] // (NW * CHUNK)
    base = wid * (n_chunks * CHUNK)

    pltpu.sync_copy(pos_hbm, pos_v)
    pltpu.sync_copy(g_hbm, g_v)
    pltpu.sync_copy(b_hbm, b_v)

    g = [g_v[pl.ds(k * L, L)] for k in range(4)]
    b = [b_v[pl.ds(k * L, L)] for k in range(4)]

    idx_bufs = (idx0, idx1)
    row_bufs = (rows0, rows1)
    sems = (sem0, sem1)

    def start_gather(c, buf):
        pltpu.sync_copy(idx_hbm.at[pl.ds(base + c * CHUNK, CHUNK)],
                        idx_bufs[buf])
        pltpu.make_async_copy(tok_hbm.at[idx_bufs[buf]], row_bufs[buf],
                              sems[buf]).start()

    start_gather(0, 0)

    def compute_chunk(c, buf):
        rows = row_bufs[buf]
        pltpu.make_async_copy(tok_hbm.at[idx_bufs[buf]], rows,
                              sems[buf]).wait()
        s0 = (c * CHUNK) % SEQ

        def row_body(i, s):
            h = []
            for k in range(4):
                t = rows[i, pl.ds(k * L, L)]
                p = pos_v[s, pl.ds(k * L, L)]
                h.append(t + p)
            hsum = (h[0] + h[1]) + (h[2] + h[3])
            hsq = (h[0] * h[0] + h[1] * h[1]) + (h[2] * h[2] + h[3] * h[3])
            tot = jnp.sum(hsum)
            totsq = jnp.sum(hsq)
            mu = jnp.full((L,), tot, jnp.float32) * (1.0 / D)
            ex2 = jnp.full((L,), totsq, jnp.float32) * (1.0 / D)
            var = ex2 - mu * mu
            r = _rsqrt(var + 1e-5)
            for k in range(4):
                rows[i, pl.ds(k * L, L)] = (h[k] - mu) * r * g[k] + b[k]
            s = s + 1
            return jnp.where(s == SEQ, 0, s)

        lax.fori_loop(0, CHUNK, row_body, s0)
        pltpu.sync_copy(rows, out_hbm.at[pl.ds(base + c * CHUNK, CHUNK)])

    def outer(o, carry):
        for bidx in range(2):
            c = o * 2 + bidx

            @pl.when(c + 1 < n_chunks)
            def _():
                start_gather(c + 1, 1 - bidx)

            compute_chunk(c, bidx)
        return carry

    lax.fori_loop(0, n_chunks // 2, outer, 0)


def kernel(x, tok_table, pos_table, gamma, beta):
    batch, seq = x.shape
    n = batch * seq
    idx = jnp.reshape(x, (n,)).astype(jnp.int32)

    mesh = plsc.VectorSubcoreMesh(core_axis_name="c", subcore_axis_name="s")
    run = pl.kernel(
        _body,
        out_type=jax.ShapeDtypeStruct((n, D), jnp.float32),
        mesh=mesh,
        scratch_types=[
            pltpu.VMEM((SEQ, D), jnp.float32),
            pltpu.VMEM((D,), jnp.float32),
            pltpu.VMEM((D,), jnp.float32),
            pltpu.VMEM((CHUNK,), jnp.int32),
            pltpu.VMEM((CHUNK,), jnp.int32),
            pltpu.VMEM((CHUNK, D), jnp.float32),
            pltpu.VMEM((CHUNK, D), jnp.float32),
            pltpu.SemaphoreType.DMA,
            pltpu.SemaphoreType.DMA,
        ],
    )
    out = run(tok_table, idx, pos_table, gamma, beta)
    return jnp.reshape(out, (batch, seq, D))


# speedup vs baseline: 1.0627x; 1.0627x over previous
import jax
import jax.numpy as jnp
from jax import lax
from jax.experimental import pallas as pl
from jax.experimental.pallas import tpu as pltpu
from jax.experimental.pallas import tpu_sc as plsc

D = 64
L = 16
NW = 32
CHUNK = 128
SEQ = 200


def _rsqrt(x):
    xi = lax.bitcast_convert_type(x, jnp.int32)
    yi = jnp.int32(0x5F3759DF) - lax.shift_right_arithmetic(xi, jnp.int32(1))
    y = lax.bitcast_convert_type(yi, jnp.float32)
    half_x = x * 0.5
    for _ in range(3):
        y = y * (1.5 - half_x * y * y)
    return y


def _body(tok_hbm, idx_hbm, pos_hbm, g_hbm, b_hbm, out_hbm,
          pos_v, g_v, b_v, idx0, idx1, rows0, rows1, sem0, sem1):
    wid = lax.axis_index("s") * 2 + lax.axis_index("c")
    n_chunks = idx_hbm.shape[0] // (NW * CHUNK)
    base = wid * (n_chunks * CHUNK)

    pltpu.sync_copy(pos_hbm, pos_v)
    pltpu.sync_copy(g_hbm, g_v)
    pltpu.sync_copy(b_hbm, b_v)

    g = [g_v[pl.ds(k * L, L)] for k in range(4)]
    b = [b_v[pl.ds(k * L, L)] for k in range(4)]

    idx_bufs = (idx0, idx1)
    row_bufs = (rows0, rows1)
    sems = (sem0, sem1)

    def start_gather(c, buf):
        pltpu.sync_copy(idx_hbm.at[pl.ds(base + c * CHUNK, CHUNK)],
                        idx_bufs[buf])
        pltpu.make_async_copy(tok_hbm.at[idx_bufs[buf]], row_bufs[buf],
                              sems[buf]).start()

    start_gather(0, 0)

    def compute_chunk(c, buf):
        rows = row_bufs[buf]
        pltpu.make_async_copy(tok_hbm.at[idx_bufs[buf]], rows,
                              sems[buf]).wait()
        s0 = (c * CHUNK) % SEQ

        def row_body(i, s):
            h = []
            for k in range(4):
                t = rows[i, pl.ds(k * L, L)]
                p = pos_v[s, pl.ds(k * L, L)]
                h.append(t + p)
            hsum = (h[0] + h[1]) + (h[2] + h[3])
            hsq = (h[0] * h[0] + h[1] * h[1]) + (h[2] * h[2] + h[3] * h[3])
            tot = jnp.sum(hsum)
            totsq = jnp.sum(hsq)
            mu = jnp.full((L,), tot, jnp.float32) * (1.0 / D)
            ex2 = jnp.full((L,), totsq, jnp.float32) * (1.0 / D)
            var = ex2 - mu * mu
            r = _rsqrt(var + 1e-5)
            for k in range(4):
                rows[i, pl.ds(k * L, L)] = (h[k] - mu) * r * g[k] + b[k]
            s = s + 1
            return jnp.where(s == SEQ, 0, s)

        lax.fori_loop(0, CHUNK, row_body, s0)
        pltpu.sync_copy(rows, out_hbm.at[pl.ds(base + c * CHUNK, CHUNK)])

    def outer(o, carry):
        for bidx in range(2):
            c = o * 2 + bidx

            @pl.when(c + 1 < n_chunks)
            def _():
                start_gather(c + 1, 1 - bidx)

            compute_chunk(c, bidx)
        return carry

    lax.fori_loop(0, n_chunks // 2, outer, 0)


def kernel(x, tok_table, pos_table, gamma, beta):
    batch, seq = x.shape
    n = batch * seq
    idx = jnp.reshape(x, (n,)).astype(jnp.int32)

    mesh = plsc.VectorSubcoreMesh(core_axis_name="c", subcore_axis_name="s")
    run = pl.kernel(
        _body,
        out_type=jax.ShapeDtypeStruct((n, D), jnp.float32),
        mesh=mesh,
        compiler_params=pltpu.CompilerParams(
            needs_layout_passes=False, use_tc_tiling_on_sc=False),
        scratch_types=[
            pltpu.VMEM((SEQ, D), jnp.float32),
            pltpu.VMEM((D,), jnp.float32),
            pltpu.VMEM((D,), jnp.float32),
            pltpu.VMEM((CHUNK,), jnp.int32),
            pltpu.VMEM((CHUNK,), jnp.int32),
            pltpu.VMEM((CHUNK, D), jnp.float32),
            pltpu.VMEM((CHUNK, D), jnp.float32),
            pltpu.SemaphoreType.DMA,
            pltpu.SemaphoreType.DMA,
        ],
    )
    out = run(tok_table, idx, pos_table, gamma, beta)
    return jnp.reshape(out, (batch, seq, D))


# speedup vs baseline: 1.0804x; 1.0167x over previous
import jax
import jax.numpy as jnp
from jax import lax
from jax.experimental import pallas as pl
from jax.experimental.pallas import tpu as pltpu
from jax.experimental.pallas import tpu_sc as plsc

D = 64
L = 16
NW = 32
CHUNK = 128
SEQ = 200


def _rsqrt(x):
    xi = lax.bitcast_convert_type(x, jnp.int32)
    yi = jnp.int32(0x5F3759DF) - lax.shift_right_arithmetic(xi, jnp.int32(1))
    y = lax.bitcast_convert_type(yi, jnp.float32)
    half_x = x * 0.5
    for _ in range(3):
        y = y * (1.5 - half_x * y * y)
    return y


def _body(tok2_hbm, idxp_hbm, off_hbm, pos_hbm, g_hbm, b_hbm, out_hbm,
          pos_v, g_v, b_v, idx0, idx1, off0, off1, rows0, rows1,
          ob0, ob1, sem0, sem1):
    wid = lax.axis_index("s") * 2 + lax.axis_index("c")
    n_chunks = idxp_hbm.shape[0] // (NW * CHUNK)
    base = wid * (n_chunks * CHUNK)

    pltpu.sync_copy(pos_hbm, pos_v)
    pltpu.sync_copy(g_hbm, g_v)
    pltpu.sync_copy(b_hbm, b_v)

    g = [g_v[pl.ds(k * L, L)] for k in range(4)]
    b = [b_v[pl.ds(k * L, L)] for k in range(4)]

    idx_bufs = (idx0, idx1)
    off_bufs = (off0, off1)
    row_bufs = (rows0, rows1)
    out_bufs = (ob0, ob1)
    sems = (sem0, sem1)

    def start_gather(c, buf):
        pltpu.sync_copy(idxp_hbm.at[pl.ds(base + c * CHUNK, CHUNK)],
                        idx_bufs[buf])
        pltpu.sync_copy(off_hbm.at[pl.ds(base + c * CHUNK, CHUNK)],
                        off_bufs[buf])
        pltpu.make_async_copy(tok2_hbm.at[idx_bufs[buf]], row_bufs[buf],
                              sems[buf]).start()

    start_gather(0, 0)

    def compute_chunk(c, buf):
        rows = row_bufs[buf]
        offs = off_bufs[buf]
        ob = out_bufs[buf]
        pltpu.make_async_copy(tok2_hbm.at[idx_bufs[buf]], rows,
                              sems[buf]).wait()
        s0 = (c * CHUNK) % SEQ

        def group_body(gidx, s):
            row0 = gidx * L
            offv = offs[pl.ds(row0, L)]
            for k in range(L):
                i = row0 + k
                off = offv[k]
                sk = s + k
                sk = jnp.where(sk >= SEQ, sk - SEQ, sk)
                h = []
                for j in range(4):
                    t = rows[i, pl.ds(off + j * L, L)]
                    p = pos_v[sk, pl.ds(j * L, L)]
                    h.append(t + p)
                hsum = (h[0] + h[1]) + (h[2] + h[3])
                hsq = (h[0] * h[0] + h[1] * h[1]) + (h[2] * h[2] + h[3] * h[3])
                tot = jnp.sum(hsum)
                totsq = jnp.sum(hsq)
                mu = jnp.full((L,), tot, jnp.float32) * (1.0 / D)
                ex2 = jnp.full((L,), totsq, jnp.float32) * (1.0 / D)
                var = ex2 - mu * mu
                r = _rsqrt(var + 1e-5)
                for j in range(4):
                    ob[i, pl.ds(j * L, L)] = (h[j] - mu) * r * g[j] + b[j]
            s = s + L
            return jnp.where(s >= SEQ, s - SEQ, s)

        lax.fori_loop(0, CHUNK // L, group_body, s0)
        pltpu.sync_copy(ob, out_hbm.at[pl.ds(base + c * CHUNK, CHUNK)])

    def outer(o, carry):
        for bidx in range(2):
            c = o * 2 + bidx

            @pl.when(c + 1 < n_chunks)
            def _():
                start_gather(c + 1, 1 - bidx)

            compute_chunk(c, bidx)
        return carry

    lax.fori_loop(0, n_chunks // 2, outer, 0)


def kernel(x, tok_table, pos_table, gamma, beta):
    batch, seq = x.shape
    n = batch * seq
    idx = jnp.reshape(x, (n,)).astype(jnp.int32)
    idx_packed = lax.shift_right_logical(idx, 1)
    half_off = lax.shift_left(jnp.bitwise_and(idx, 1), 6)
    tok2 = jnp.reshape(tok_table, (tok_table.shape[0] // 2, 2 * D))

    mesh = plsc.VectorSubcoreMesh(core_axis_name="c", subcore_axis_name="s")
    run = pl.kernel(
        _body,
        out_type=jax.ShapeDtypeStruct((n, D), jnp.float32),
        mesh=mesh,
        compiler_params=pltpu.CompilerParams(needs_layout_passes=False),
        scratch_types=[
            pltpu.VMEM((SEQ, D), jnp.float32),
            pltpu.VMEM((D,), jnp.float32),
            pltpu.VMEM((D,), jnp.float32),
            pltpu.VMEM((CHUNK,), jnp.int32),
            pltpu.VMEM((CHUNK,), jnp.int32),
            pltpu.VMEM((CHUNK,), jnp.int32),
            pltpu.VMEM((CHUNK,), jnp.int32),
            pltpu.VMEM((CHUNK, 2 * D), jnp.float32),
            pltpu.VMEM((CHUNK, 2 * D), jnp.float32),
            pltpu.VMEM((CHUNK, D), jnp.float32),
            pltpu.VMEM((CHUNK, D), jnp.float32),
            pltpu.SemaphoreType.DMA,
            pltpu.SemaphoreType.DMA,
        ],
    )
    out = run(tok2, idx_packed, half_off, pos_table, gamma, beta)
    return jnp.reshape(out, (batch, seq, D))
